# baseline (speedup 1.0000x reference)
import functools

import jax
import jax.numpy as jnp
from jax import lax
from jax.experimental import pallas as pl
from jax.experimental.pallas import tpu as pltpu
from jax.experimental.pallas import tpu_sc as plsc

_BATCH, _SEQ = 16384, 50
_D = 64
_B = _BATCH * _SEQ
_NC, _NS = 2, 16
_NW = _NC * _NS
_BPW = _B // _NW
_CHUNK = 128
_NCHUNK = _BPW // _CHUNK
_NBUF = 8

_mesh = plsc.VectorSubcoreMesh(core_axis_name="c", subcore_axis_name="s")


@functools.partial(
    pl.kernel,
    mesh=_mesh,
    out_type=jax.ShapeDtypeStruct((_B, _D), jnp.float32),
    scratch_types=[
        pltpu.VMEM((_NCHUNK, _CHUNK), jnp.int32),
        [pltpu.VMEM((_CHUNK, _D), jnp.float32) for _ in range(_NBUF)],
        [pltpu.SemaphoreType.DMA for _ in range(_NBUF)],
        [pltpu.SemaphoreType.DMA for _ in range(_NBUF)],
    ],
    compiler_params=pltpu.CompilerParams(use_tc_tiling_on_sc=False),
)
def _embed(x_hbm, table_hbm, out_hbm, idx_v, rows, gsem, ssem):
    wid = lax.axis_index("s") * _NC + lax.axis_index("c")
    base = wid * _BPW

    pltpu.sync_copy(x_hbm.at[wid], idx_v)

    def gather_desc(c, b):
        return pltpu.make_async_copy(
            table_hbm.at[idx_v.at[c]], rows[b], gsem[b])

    def store_desc(c, b):
        return pltpu.make_async_copy(
            rows[b], out_hbm.at[pl.ds(base + c * _CHUNK, _CHUNK)], ssem[b])

    for b in range(_NBUF - 1):
        gather_desc(b, b).start()

    def group(g, carry):
        for b in range(_NBUF):
            c = g * _NBUF + b
            gather_desc(c, b).wait()
            store_desc(c, b).start()
            nb = (b + _NBUF - 1) % _NBUF

            @pl.when(c >= 1)
            def _():
                store_desc(c - 1, nb).wait()

            @pl.when(c + _NBUF - 1 < _NCHUNK)
            def _():
                gather_desc(c + _NBUF - 1, nb).start()
        return carry

    lax.fori_loop(0, _NCHUNK // _NBUF, group, 0)
    store_desc(_NCHUNK - 1, (_NCHUNK - 1) % _NBUF).wait()


def kernel(x, table):
    xi = x.reshape(_NW, _NCHUNK, _CHUNK).astype(jnp.int32)
    out = _embed(xi, table)
    return out.reshape(_BATCH, _SEQ, _D)
